# baseline (speedup 1.0000x reference)
import jax
import jax.numpy as jnp
from jax import lax
from jax.experimental import pallas as pl
from jax.experimental.pallas import tpu as pltpu
from jax.experimental.pallas import tpu_sc as plsc

_L = 16
_NC, _NS = 2, 16
_NW = _NC * _NS
_RC = 28
_NCK = 7
_NBUF = 3
_GPP = 24


def _sc_body(x_hbm, tp_hbm, o_hbm, tp_v, xo_v, av_v,
             tp_sem, in_s0, in_s1, in_s2, out_s0, out_s1, out_s2):
    p = 196
    c = 768
    pp = p * p
    bpw = (x_hbm.shape[0] // (p * c)) // _NW
    in_sems = (in_s0, in_s1, in_s2)
    out_sems = (out_s0, out_s1, out_s2)
    cid = lax.axis_index("c")
    sid = lax.axis_index("s")
    wid = sid * _NC + cid
    cw = _RC * c

    units = [(bi, ph, ck)
             for bi in range(bpw) for ph in range(3) for ck in range(_NCK)]
    nunits = len(units)

    def chunk_src(u):
        bi, _, ck = units[u]
        base = (wid * bpw + bi) * p * c + ck * cw
        return x_hbm.at[pl.ds(base, cw)]

    def start_in(u):
        return pltpu.async_copy(chunk_src(u), xo_v.at[u % _NBUF],
                                in_sems[u % _NBUF])

    def start_out(u):
        bi, _, ck = units[u]
        base = (wid * bpw + bi) * p * c + ck * cw
        return pltpu.async_copy(xo_v.at[u % _NBUF],
                                o_hbm.at[pl.ds(base, cw)],
                                out_sems[u % _NBUF])

    def amax_chunk(buf, ck, half, carry):
        xo = xo_v.at[buf]
        ch0 = half * _GPP * _L

        def body(q, carry):
            ms, idxs = carry
            qg = ck * _RC + q
            nms, nidxs = [], []
            for l in range(_GPP):
                v = xo[pl.ds(q * c + ch0 + l * _L, _L)]
                gt = v > ms[l]
                nms.append(jnp.where(gt, v, ms[l]))
                nidxs.append(jnp.where(gt, qg, idxs[l]))
            return tuple(nms), tuple(nidxs)

        return lax.fori_loop(0, _RC, body, carry)

    def gather_chunk(buf, ck):
        xo = xo_v.at[buf]
        q0 = ck * _RC

        def per_group(l, _):
            lw = l * _L
            av0 = av_v[pl.ds(lw, _L)] + q0
            t0 = plsc.load_gather(tp_v, [av0])
            x0 = xo[pl.ds(lw, _L)]

            def gbody(q, carry):
                av, tprev, xprev = carry
                av = av + 1
                t = plsc.load_gather(tp_v, [av])
                xq = xo[pl.ds(q * c + lw, _L)]
                xo[pl.ds((q - 1) * c + lw, _L)] = jnp.maximum(
                    xprev * tprev, 0.0)
                return av, t, xq

            _, tl, xl = lax.fori_loop(1, _RC, gbody, (av0, t0, x0),
                                      unroll=4)
            xo[pl.ds((_RC - 1) * c + lw, _L)] = jnp.maximum(xl * tl, 0.0)
            return 0

        lax.fori_loop(0, c // _L, per_group, 0)

    in_h = [None] * nunits
    out_h = [None] * nunits
    tp_h = None
    for u in range(min(2, nunits)):
        in_h[u] = start_in(u)
    neg = jnp.full((_L,), -jnp.inf, jnp.float32)
    zer = jnp.zeros((_L,), jnp.int32)
    carry = None
    for u in range(nunits):
        bi, ph, ck = units[u]
        if ph == 0 and ck == 0:
            tp_h = pltpu.async_copy(
                tp_hbm.at[pl.ds((wid * bpw + bi) * pp, pp)], tp_v, tp_sem)
        if ph < 2 and ck == 0:
            carry = (tuple([neg] * _GPP), tuple([zer] * _GPP))
        in_h[u].wait()
        if ph < 2:
            carry = amax_chunk(u % _NBUF, ck, ph, carry)
            if ck == _NCK - 1:
                _, idxs = carry
                for l in range(_GPP):
                    g = ph * _GPP + l
                    av_v[pl.ds(g * _L, _L)] = idxs[l] * p
            if ph == 1 and ck == _NCK - 1:
                tp_h.wait()
        else:
            gather_chunk(u % _NBUF, ck)
            out_h[u] = start_out(u)
        if u + 2 < nunits:
            v = u - 1
            if v >= 0 and units[v][1] == 2:
                out_h[v].wait()
            in_h[u + 2] = start_in(u + 2)
    for u in range(max(0, nunits - 3), nunits):
        if units[u][1] == 2 and out_h[u] is not None:
            out_h[u].wait()


def kernel(input, t_p):
    b, h, w, c = input.shape
    p = h * w
    x1 = input.reshape(b * p * c)
    tp1 = t_p.reshape(b * p * p)
    mesh = plsc.VectorSubcoreMesh(core_axis_name="c", subcore_axis_name="s")
    run = pl.kernel(
        _sc_body,
        out_type=jax.ShapeDtypeStruct((b * p * c,), jnp.float32),
        mesh=mesh,
        compiler_params=pltpu.CompilerParams(use_tc_tiling_on_sc=False,
                                             needs_layout_passes=False),
        scratch_types=[
            pltpu.VMEM((p * p,), jnp.float32),
            pltpu.VMEM((_NBUF, _RC * c), jnp.float32),
            pltpu.VMEM((c,), jnp.int32),
            pltpu.SemaphoreType.DMA,
            pltpu.SemaphoreType.DMA,
            pltpu.SemaphoreType.DMA,
            pltpu.SemaphoreType.DMA,
            pltpu.SemaphoreType.DMA,
            pltpu.SemaphoreType.DMA,
            pltpu.SemaphoreType.DMA,
        ],
    )
    out = run(x1, tp1)
    return out.reshape(b, h, w, c)


# speedup vs baseline: 1.2389x; 1.2389x over previous
import jax
import jax.numpy as jnp
from jax import lax
from jax.experimental import pallas as pl
from jax.experimental.pallas import tpu as pltpu
from jax.experimental.pallas import tpu_sc as plsc

_L = 16
_NC, _NS = 2, 16
_NW = _NC * _NS
_SLAB = 128
_NBUF = 3


def _compute_slab(tp_v, xo, p):
    def per_group(l, _):
        lane = pl.ds(l * _L, _L)

        def amax(q, carry):
            m, idx = carry
            v = xo[q, lane]
            gt = v > m
            return jnp.where(gt, v, m), jnp.where(gt, q, idx)

        m0 = xo[0, lane]
        idx0 = jnp.zeros((_L,), jnp.int32)
        _, idx = lax.fori_loop(1, p, amax, (m0, idx0), unroll=8)

        av0 = idx * p
        t0 = plsc.load_gather(tp_v, [av0])
        x0 = xo[0, lane]

        def gpass(q, carry):
            av, tprev, xprev = carry
            av = av + 1
            t = plsc.load_gather(tp_v, [av])
            xq = xo[q, lane]
            xo[q - 1, lane] = jnp.maximum(xprev * tprev, 0.0)
            return av, t, xq

        _, tl, xl = lax.fori_loop(1, p, gpass, (av0, t0, x0), unroll=8)
        xo[p - 1, lane] = jnp.maximum(xl * tl, 0.0)
        return 0

    lax.fori_loop(0, _SLAB // _L, per_group, 0)


def _sc_body(x_hbm, tp_hbm, o_hbm, tp_v, xo_v,
             in_s0, in_s1, in_s2, out_s0, out_s1, out_s2):
    b_total, p, c = x_hbm.shape
    bpw = b_total // _NW
    nslab = c // _SLAB
    nunits = bpw * nslab
    in_sems = (in_s0, in_s1, in_s2)
    out_sems = (out_s0, out_s1, out_s2)
    cid = lax.axis_index("c")
    sid = lax.axis_index("s")
    wid = sid * _NC + cid

    def unit_batch(u):
        return wid * bpw + u // nslab

    def start_in(u):
        g = u % nslab
        return pltpu.async_copy(
            x_hbm.at[unit_batch(u), :, pl.ds(g * _SLAB, _SLAB)],
            xo_v.at[u % _NBUF], in_sems[u % _NBUF])

    def start_out(u):
        g = u % nslab
        return pltpu.async_copy(
            xo_v.at[u % _NBUF],
            o_hbm.at[unit_batch(u), :, pl.ds(g * _SLAB, _SLAB)],
            out_sems[u % _NBUF])

    in_h = [None] * nunits
    out_h = [None] * nunits
    for u in range(min(2, nunits)):
        in_h[u] = start_in(u)
    for u in range(nunits):
        if u % nslab == 0:
            pltpu.sync_copy(tp_hbm.at[unit_batch(u)], tp_v)
        in_h[u].wait()
        _compute_slab(tp_v, xo_v.at[u % _NBUF], p)
        out_h[u] = start_out(u)
        if u + 2 < nunits:
            if u >= 1:
                out_h[u - 1].wait()
            in_h[u + 2] = start_in(u + 2)
    for u in range(max(0, nunits - 2), nunits):
        out_h[u].wait()


def kernel(input, t_p):
    b, h, w, c = input.shape
    p = h * w
    x = input.reshape(b, p, c)
    tp = t_p.reshape(b, p * p)
    mesh = plsc.VectorSubcoreMesh(core_axis_name="c", subcore_axis_name="s")
    run = pl.kernel(
        _sc_body,
        out_type=jax.ShapeDtypeStruct((b, p, c), jnp.float32),
        mesh=mesh,
        compiler_params=pltpu.CompilerParams(use_tc_tiling_on_sc=False,
                                             needs_layout_passes=False),
        scratch_types=[
            pltpu.VMEM((p * p,), jnp.float32),
            pltpu.VMEM((_NBUF, p, _SLAB), jnp.float32),
            pltpu.SemaphoreType.DMA,
            pltpu.SemaphoreType.DMA,
            pltpu.SemaphoreType.DMA,
            pltpu.SemaphoreType.DMA,
            pltpu.SemaphoreType.DMA,
            pltpu.SemaphoreType.DMA,
        ],
    )
    out = run(x, tp)
    return out.reshape(b, h, w, c)


# speedup vs baseline: 1.3756x; 1.1103x over previous
import jax
import jax.numpy as jnp
from jax import lax
from jax.experimental import pallas as pl
from jax.experimental.pallas import tpu as pltpu
from jax.experimental.pallas import tpu_sc as plsc

_L = 16
_NC, _NS = 2, 16
_NW = _NC * _NS
_SLAB = 128
_NBUF = 3


def _compute_slab(tp_v, xo, p):
    def per_pair(pr, _):
        la = pl.ds((pr * 2) * _L, _L)
        lb = pl.ds((pr * 2 + 1) * _L, _L)

        def amax(q, carry):
            ma, ia, mb, ib = carry
            va = xo[q, la]
            vb = xo[q, lb]
            ga = va > ma
            gb = vb > mb
            return (jnp.where(ga, va, ma), jnp.where(ga, q, ia),
                    jnp.where(gb, vb, mb), jnp.where(gb, q, ib))

        z = jnp.zeros((_L,), jnp.int32)
        _, ia, _, ib = lax.fori_loop(
            1, p, amax, (xo[0, la], z, xo[0, lb], z), unroll=4)

        aa0 = ia * p
        ab0 = ib * p
        ta0 = plsc.load_gather(tp_v, [aa0])
        tb0 = plsc.load_gather(tp_v, [ab0])
        xa0 = xo[0, la]
        xb0 = xo[0, lb]

        def gpass(q, carry):
            aa, ta, xa, ab, tb, xb = carry
            aa = aa + 1
            ab = ab + 1
            nta = plsc.load_gather(tp_v, [aa])
            ntb = plsc.load_gather(tp_v, [ab])
            nxa = xo[q, la]
            nxb = xo[q, lb]
            xo[q - 1, la] = jnp.maximum(xa * ta, 0.0)
            xo[q - 1, lb] = jnp.maximum(xb * tb, 0.0)
            return aa, nta, nxa, ab, ntb, nxb

        _, ta, xa, _, tb, xb = lax.fori_loop(
            1, p, gpass, (aa0, ta0, xa0, ab0, tb0, xb0), unroll=4)
        xo[p - 1, la] = jnp.maximum(xa * ta, 0.0)
        xo[p - 1, lb] = jnp.maximum(xb * tb, 0.0)
        return 0

    lax.fori_loop(0, _SLAB // (2 * _L), per_pair, 0)


def _sc_body(x_hbm, tp_hbm, o_hbm, tp_v, xo_v,
             in_s0, in_s1, in_s2, out_s0, out_s1, out_s2):
    b_total, p, c = x_hbm.shape
    bpw = b_total // _NW
    nslab = c // _SLAB
    nunits = bpw * nslab
    in_sems = (in_s0, in_s1, in_s2)
    out_sems = (out_s0, out_s1, out_s2)
    cid = lax.axis_index("c")
    sid = lax.axis_index("s")
    wid = sid * _NC + cid

    def unit_batch(u):
        return wid * bpw + u // nslab

    def start_in(u):
        g = u % nslab
        return pltpu.async_copy(
            x_hbm.at[unit_batch(u), :, pl.ds(g * _SLAB, _SLAB)],
            xo_v.at[u % _NBUF], in_sems[u % _NBUF])

    def start_out(u):
        g = u % nslab
        return pltpu.async_copy(
            xo_v.at[u % _NBUF],
            o_hbm.at[unit_batch(u), :, pl.ds(g * _SLAB, _SLAB)],
            out_sems[u % _NBUF])

    in_h = [None] * nunits
    out_h = [None] * nunits
    for u in range(min(2, nunits)):
        in_h[u] = start_in(u)
    for u in range(nunits):
        if u % nslab == 0:
            pltpu.sync_copy(tp_hbm.at[unit_batch(u)], tp_v)
        in_h[u].wait()
        _compute_slab(tp_v, xo_v.at[u % _NBUF], p)
        out_h[u] = start_out(u)
        if u + 2 < nunits:
            if u >= 1:
                out_h[u - 1].wait()
            in_h[u + 2] = start_in(u + 2)
    for u in range(max(0, nunits - 2), nunits):
        out_h[u].wait()


def kernel(input, t_p):
    b, h, w, c = input.shape
    p = h * w
    x = input.reshape(b, p, c)
    tp = t_p.reshape(b, p * p)
    mesh = plsc.VectorSubcoreMesh(core_axis_name="c", subcore_axis_name="s")
    run = pl.kernel(
        _sc_body,
        out_type=jax.ShapeDtypeStruct((b, p, c), jnp.float32),
        mesh=mesh,
        compiler_params=pltpu.CompilerParams(use_tc_tiling_on_sc=False,
                                             needs_layout_passes=False),
        scratch_types=[
            pltpu.VMEM((p * p,), jnp.float32),
            pltpu.VMEM((_NBUF, p, _SLAB), jnp.float32),
            pltpu.SemaphoreType.DMA,
            pltpu.SemaphoreType.DMA,
            pltpu.SemaphoreType.DMA,
            pltpu.SemaphoreType.DMA,
            pltpu.SemaphoreType.DMA,
            pltpu.SemaphoreType.DMA,
        ],
    )
    out = run(x, tp)
    return out.reshape(b, h, w, c)
